# Initial kernel scaffold; baseline (speedup 1.0000x reference)
import dataclasses
import functools

import jax
import jax.numpy as jnp
from jax import lax
from jax.experimental import pallas as pl
from jax.experimental.pallas import tpu as pltpu
from jax.experimental.pallas import tpu_sc as plsc

N = 10000
E = 320000
NC, NS = 2, 16
NW = NC * NS
EPW = E // NW
CHUNK = 2000
L = 16

_mesh = plsc.VectorSubcoreMesh(
    core_axis_name="c", subcore_axis_name="s", num_cores=NC, num_subcores=NS)

_cp = pltpu.CompilerParams()
if "needs_layout_passes" in pltpu.CompilerParams.__dataclass_fields__:
    _cp = dataclasses.replace(_cp, needs_layout_passes=False)


def _wid():
    return lax.axis_index("c") * NS + lax.axis_index("s")


@functools.partial(
    pl.kernel,
    out_type=jax.ShapeDtypeStruct((NW, N), jnp.float32),
    mesh=_mesh,
    compiler_params=_cp,
    name="sc_deg",
    scratch_types=[
        pltpu.VMEM((N,), jnp.float32),
        pltpu.VMEM((CHUNK,), jnp.int32),
        pltpu.VMEM((CHUNK,), jnp.float32),
    ],
)
def _sc_deg(dst_hbm, ew_hbm, zero_hbm, out_hbm, deg_v, dst_c, ew_c):
    base = _wid() * EPW
    pltpu.sync_copy(zero_hbm, deg_v)

    @pl.loop(0, EPW, step=CHUNK)
    def _(c):
        pltpu.sync_copy(dst_hbm.at[pl.ds(base + c, CHUNK)], dst_c)
        pltpu.sync_copy(ew_hbm.at[pl.ds(base + c, CHUNK)], ew_c)

        @pl.loop(0, CHUNK, step=L)
        def _(g):
            d16 = dst_c[pl.ds(g, L)]
            w16 = ew_c[pl.ds(g, L)]
            plsc.addupdate_scatter(deg_v, [d16], w16)

    pltpu.sync_copy(deg_v, out_hbm.at[_wid()])


def _make_sc_agg(F, G):
    n_pass = F // G

    @functools.partial(
        pl.kernel,
        out_type=jax.ShapeDtypeStruct((NW, F, N), jnp.float32),
        mesh=_mesh,
        compiler_params=_cp,
        name=f"sc_agg{F}",
        scratch_types=[
            pltpu.VMEM((N,), jnp.float32),
            pltpu.VMEM((EPW,), jnp.float32),
            pltpu.VMEM((G, N), jnp.float32),
            pltpu.VMEM((G, N), jnp.float32),
            pltpu.VMEM((CHUNK,), jnp.int32),
            pltpu.VMEM((CHUNK,), jnp.int32),
            pltpu.VMEM((CHUNK,), jnp.float32),
        ],
    )
    def _sc_agg(src_hbm, dst_hbm, ew_hbm, hT_hbm, dis_hbm, zero_hbm, out_hbm,
                dis_v, norm_v, h_v, acc_v, src_c, dst_c, ew_c):
        base = _wid() * EPW
        pltpu.sync_copy(dis_hbm, dis_v)

        @pl.loop(0, EPW, step=CHUNK)
        def _(c):
            pltpu.sync_copy(src_hbm.at[pl.ds(base + c, CHUNK)], src_c)
            pltpu.sync_copy(dst_hbm.at[pl.ds(base + c, CHUNK)], dst_c)
            pltpu.sync_copy(ew_hbm.at[pl.ds(base + c, CHUNK)], ew_c)

            @pl.loop(0, CHUNK, step=L)
            def _(g):
                s16 = src_c[pl.ds(g, L)]
                d16 = dst_c[pl.ds(g, L)]
                dsv = plsc.load_gather(dis_v, [s16])
                ddv = plsc.load_gather(dis_v, [d16])
                norm_v[pl.ds(c + g, L)] = dsv * ew_c[pl.ds(g, L)] * ddv

        for p in range(n_pass):
            pltpu.sync_copy(zero_hbm.at[pl.ds(0, G)], acc_v)
            pltpu.sync_copy(hT_hbm.at[pl.ds(p * G, G)], h_v)

            @pl.loop(0, EPW, step=CHUNK)
            def _(c):
                pltpu.sync_copy(src_hbm.at[pl.ds(base + c, CHUNK)], src_c)
                pltpu.sync_copy(dst_hbm.at[pl.ds(base + c, CHUNK)], dst_c)

                @pl.loop(0, CHUNK, step=L)
                def _(g):
                    s16 = src_c[pl.ds(g, L)]
                    d16 = dst_c[pl.ds(g, L)]
                    n16 = norm_v[pl.ds(c + g, L)]
                    for f in range(G):
                        fidx = jnp.full((L,), f, jnp.int32)
                        hf = plsc.load_gather(h_v, [fidx, s16])
                        plsc.addupdate_scatter(acc_v, [fidx, d16], hf * n16)

            pltpu.sync_copy(acc_v, out_hbm.at[_wid(), pl.ds(p * G, G)])

    return _sc_agg


_sc_agg16 = _make_sc_agg(16, 4)
_sc_agg2 = _make_sc_agg(2, 2)


def _tc_h1T(W1, x):
    bn = 2000

    def body(w_ref, x_ref, o_ref):
        o_ref[...] = lax.dot_general(
            w_ref[...], x_ref[...], (((0,), (1,)), ((), ())),
            preferred_element_type=jnp.float32)

    return pl.pallas_call(
        body,
        grid=(N // bn,),
        in_specs=[
            pl.BlockSpec((128, 16), lambda i: (0, 0)),
            pl.BlockSpec((bn, 128), lambda i: (i, 0)),
        ],
        out_specs=pl.BlockSpec((16, bn), lambda i: (0, i)),
        out_shape=jax.ShapeDtypeStruct((16, N), jnp.float32),
    )(W1, x)


def _tc_dis(deg_part):
    def body(dp_ref, o_ref):
        deg = jnp.sum(dp_ref[...], axis=0, keepdims=True) + 1.0
        o_ref[...] = lax.rsqrt(deg)

    return pl.pallas_call(
        body, out_shape=jax.ShapeDtypeStruct((1, N), jnp.float32),
    )(deg_part)


def _tc_layer2_prep(agg1p, h1T, dis2d, b1c, W2):
    def body(a_ref, h_ref, d_ref, b_ref, w_ref, o_ref):
        aggsum = jnp.sum(a_ref[...], axis=0)
        dis2 = d_ref[...] * d_ref[...]
        out1 = aggsum + h_ref[...] * dis2 + b_ref[...]
        r = jnp.maximum(out1, 0.0)
        o_ref[...] = lax.dot_general(
            w_ref[...], r, (((0,), (0,)), ((), ())),
            preferred_element_type=jnp.float32)

    return pl.pallas_call(
        body, out_shape=jax.ShapeDtypeStruct((2, N), jnp.float32),
    )(agg1p, h1T, dis2d, b1c, W2)


def _tc_final(agg2p, h2T, dis2d, b2c):
    def body(a_ref, h_ref, d_ref, b_ref, o_ref):
        aggsum = jnp.sum(a_ref[...], axis=0)
        dis2 = d_ref[...] * d_ref[...]
        o2 = aggsum + h_ref[...] * dis2 + b_ref[...]
        m = jnp.max(o2, axis=0, keepdims=True)
        lse = m + jnp.log(jnp.sum(jnp.exp(o2 - m), axis=0, keepdims=True))
        o_ref[...] = o2 - lse

    return pl.pallas_call(
        body, out_shape=jax.ShapeDtypeStruct((2, N), jnp.float32),
    )(agg2p, h2T, dis2d, b2c)


@jax.jit
def kernel(x, edge_index, edge_weight, W1, b1, W2, b2):
    ei = edge_index.astype(jnp.int32)
    src = ei[0]
    dst = ei[1]
    zeros4 = jnp.zeros((4, N), jnp.float32)
    zeros1 = jnp.zeros((N,), jnp.float32)

    deg_part = _sc_deg(dst, edge_weight, zeros1)
    h1T = _tc_h1T(W1, x)
    dis2d = _tc_dis(deg_part)
    dis1d = dis2d.reshape(N)

    agg1p = _sc_agg16(src, dst, edge_weight, h1T, dis1d, zeros4)
    h2T = _tc_layer2_prep(agg1p, h1T, dis2d, b1.reshape(16, 1), W2)
    agg2p = _sc_agg2(src, dst, edge_weight, h2T, dis1d, zeros4[:2])
    lsmT = _tc_final(agg2p, h2T, dis2d, b2.reshape(2, 1))
    return lsmT.T


# speedup vs baseline: 35.4357x; 35.4357x over previous
import dataclasses
import functools

import jax
import jax.numpy as jnp
from jax import lax
from jax.experimental import pallas as pl
from jax.experimental.pallas import tpu as pltpu
from jax.experimental.pallas import tpu_sc as plsc

N = 10000
E = 320000
NC, NS = 2, 16
NW = NC * NS
EPW = E // NW
CHUNK = 2000
L = 16

_mesh = plsc.VectorSubcoreMesh(
    core_axis_name="c", subcore_axis_name="s", num_cores=NC, num_subcores=NS)

_cp = pltpu.CompilerParams()
if "needs_layout_passes" in pltpu.CompilerParams.__dataclass_fields__:
    _cp = dataclasses.replace(_cp, needs_layout_passes=False)


def _wid():
    return lax.axis_index("c") * NS + lax.axis_index("s")


@functools.partial(
    pl.kernel,
    out_type=jax.ShapeDtypeStruct((NW, N), jnp.float32),
    mesh=_mesh,
    compiler_params=_cp,
    name="sc_deg",
    scratch_types=[
        pltpu.VMEM((N,), jnp.float32),
        pltpu.VMEM((CHUNK,), jnp.int32),
        pltpu.VMEM((CHUNK,), jnp.float32),
    ],
)
def _sc_deg(dst_hbm, ew_hbm, zero_hbm, out_hbm, deg_v, dst_c, ew_c):
    base = _wid() * EPW
    pltpu.sync_copy(zero_hbm, deg_v)

    @pl.loop(0, EPW, step=CHUNK)
    def _(c):
        pltpu.sync_copy(dst_hbm.at[pl.ds(base + c, CHUNK)], dst_c)
        pltpu.sync_copy(ew_hbm.at[pl.ds(base + c, CHUNK)], ew_c)

        @pl.loop(0, CHUNK, step=L)
        def _(g):
            d16 = dst_c[pl.ds(g, L)]
            w16 = ew_c[pl.ds(g, L)]
            plsc.addupdate_scatter(deg_v, [d16], w16)

    pltpu.sync_copy(deg_v, out_hbm.at[_wid()])


def _make_sc_agg(F, G):
    n_pass = F // G

    @functools.partial(
        pl.kernel,
        out_type=jax.ShapeDtypeStruct((NW, F, N), jnp.float32),
        mesh=_mesh,
        compiler_params=_cp,
        name=f"sc_agg{F}",
        scratch_types=[
            pltpu.VMEM((N,), jnp.float32),
            pltpu.VMEM((EPW,), jnp.float32),
            pltpu.VMEM((G, N), jnp.float32),
            pltpu.VMEM((G, N), jnp.float32),
            pltpu.VMEM((CHUNK,), jnp.int32),
            pltpu.VMEM((CHUNK,), jnp.int32),
            pltpu.VMEM((CHUNK,), jnp.float32),
        ],
    )
    def _sc_agg(src_hbm, dst_hbm, ew_hbm, hT_hbm, dis_hbm, zero_hbm, out_hbm,
                dis_v, norm_v, h_v, acc_v, src_c, dst_c, ew_c):
        base = _wid() * EPW
        pltpu.sync_copy(dis_hbm, dis_v)

        @pl.loop(0, EPW, step=CHUNK)
        def _(c):
            pltpu.sync_copy(src_hbm.at[pl.ds(base + c, CHUNK)], src_c)
            pltpu.sync_copy(dst_hbm.at[pl.ds(base + c, CHUNK)], dst_c)
            pltpu.sync_copy(ew_hbm.at[pl.ds(base + c, CHUNK)], ew_c)

            @pl.loop(0, CHUNK, step=L)
            def _(g):
                s16 = src_c[pl.ds(g, L)]
                d16 = dst_c[pl.ds(g, L)]
                dsv = plsc.load_gather(dis_v, [s16])
                ddv = plsc.load_gather(dis_v, [d16])
                norm_v[pl.ds(c + g, L)] = dsv * ew_c[pl.ds(g, L)] * ddv

        for p in range(n_pass):
            pltpu.sync_copy(zero_hbm.at[pl.ds(0, G)], acc_v)
            pltpu.sync_copy(hT_hbm.at[pl.ds(p * G, G)], h_v)

            @pl.loop(0, EPW, step=CHUNK)
            def _(c):
                pltpu.sync_copy(src_hbm.at[pl.ds(base + c, CHUNK)], src_c)
                pltpu.sync_copy(dst_hbm.at[pl.ds(base + c, CHUNK)], dst_c)

                @pl.loop(0, CHUNK, step=L)
                def _(g):
                    s16 = src_c[pl.ds(g, L)]
                    d16 = dst_c[pl.ds(g, L)]
                    n16 = norm_v[pl.ds(c + g, L)]
                    for f in range(G):
                        fidx = jnp.full((L,), f, jnp.int32)
                        hf = plsc.load_gather(h_v, [fidx, s16])
                        plsc.addupdate_scatter(acc_v, [fidx, d16], hf * n16)

            pltpu.sync_copy(acc_v, out_hbm.at[_wid(), pl.ds(p * G, G)])

    return _sc_agg


_sc_agg16 = _make_sc_agg(16, 4)
_sc_agg2 = _make_sc_agg(2, 2)


def _tc_h1T(W1, x):
    def body(w_ref, x_ref, o_ref):
        o_ref[...] = lax.dot_general(
            w_ref[...], x_ref[...], (((0,), (1,)), ((), ())),
            preferred_element_type=jnp.float32)

    return pl.pallas_call(
        body, out_shape=jax.ShapeDtypeStruct((16, N), jnp.float32),
    )(W1, x)


def _tc_dis(deg_part):
    def body(dp_ref, o_ref):
        deg = jnp.sum(dp_ref[...], axis=0, keepdims=True) + 1.0
        o_ref[...] = lax.rsqrt(deg)

    return pl.pallas_call(
        body, out_shape=jax.ShapeDtypeStruct((1, N), jnp.float32),
    )(deg_part)


def _tc_layer2_prep(agg1p, h1T, dis2d, b1c, W2):
    def body(a_ref, h_ref, d_ref, b_ref, w_ref, o_ref):
        aggsum = jnp.sum(a_ref[...], axis=0)
        dis2 = d_ref[...] * d_ref[...]
        out1 = aggsum + h_ref[...] * dis2 + b_ref[...]
        r = jnp.maximum(out1, 0.0)
        o_ref[...] = lax.dot_general(
            w_ref[...], r, (((0,), (0,)), ((), ())),
            preferred_element_type=jnp.float32)

    return pl.pallas_call(
        body, out_shape=jax.ShapeDtypeStruct((2, N), jnp.float32),
    )(agg1p, h1T, dis2d, b1c, W2)


def _tc_final(agg2p, h2T, dis2d, b2c):
    def body(a_ref, h_ref, d_ref, b_ref, o_ref):
        aggsum = jnp.sum(a_ref[...], axis=0)
        dis2 = d_ref[...] * d_ref[...]
        o2 = aggsum + h_ref[...] * dis2 + b_ref[...]
        m = jnp.max(o2, axis=0, keepdims=True)
        lse = m + jnp.log(jnp.sum(jnp.exp(o2 - m), axis=0, keepdims=True))
        o_ref[...] = o2 - lse

    return pl.pallas_call(
        body, out_shape=jax.ShapeDtypeStruct((2, N), jnp.float32),
    )(agg2p, h2T, dis2d, b2c)


@jax.jit
def kernel(x, edge_index, edge_weight, W1, b1, W2, b2):
    ei = edge_index.astype(jnp.int32)
    src = ei[0]
    dst = ei[1]
    zeros4 = jnp.zeros((4, N), jnp.float32)
    zeros1 = jnp.zeros((N,), jnp.float32)

    deg_part = _sc_deg(dst, edge_weight, zeros1)
    h1T = _tc_h1T(W1, x)
    dis2d = _tc_dis(deg_part)
    dis1d = dis2d.reshape(N)

    agg1p = _sc_agg16(src, dst, edge_weight, h1T, dis1d, zeros4)
    h2T = _tc_layer2_prep(agg1p, h1T, dis2d, b1.reshape(16, 1), W2)
    agg2p = _sc_agg2(src, dst, edge_weight, h2T, dis1d, zeros4[:2])
    lsmT = _tc_final(agg2p, h2T, dis2d, b2.reshape(2, 1))
    return lsmT.T


# speedup vs baseline: 65.8490x; 1.8583x over previous
import dataclasses
import functools

import jax
import jax.numpy as jnp
from jax import lax
from jax.experimental import pallas as pl
from jax.experimental.pallas import tpu as pltpu
from jax.experimental.pallas import tpu_sc as plsc

N = 10000
E = 320000
NC, NS = 2, 16
NW = NC * NS
EPW = E // NW
CHUNK = 2000
L = 16

_mesh = plsc.VectorSubcoreMesh(
    core_axis_name="c", subcore_axis_name="s", num_cores=NC, num_subcores=NS)

_cp = pltpu.CompilerParams()
if "needs_layout_passes" in pltpu.CompilerParams.__dataclass_fields__:
    _cp = dataclasses.replace(_cp, needs_layout_passes=False)


def _wid():
    return lax.axis_index("c") * NS + lax.axis_index("s")


@functools.partial(
    pl.kernel,
    out_type=jax.ShapeDtypeStruct((NW, N), jnp.float32),
    mesh=_mesh,
    compiler_params=_cp,
    name="sc_deg",
    scratch_types=[
        pltpu.VMEM((N,), jnp.float32),
        pltpu.VMEM((EPW,), jnp.int32),
        pltpu.VMEM((EPW,), jnp.float32),
    ],
)
def _sc_deg(dst_hbm, ew_hbm, out_hbm, deg_v, dst_v, ew_v):
    base = _wid() * EPW
    pltpu.sync_copy(dst_hbm.at[pl.ds(base, EPW)], dst_v)
    pltpu.sync_copy(ew_hbm.at[pl.ds(base, EPW)], ew_v)

    z16 = jnp.zeros((L,), jnp.float32)

    @plsc.parallel_loop(0, N, step=L, unroll=8)
    def _(i):
        deg_v[pl.ds(i, L)] = z16

    @plsc.parallel_loop(0, EPW, step=L, unroll=4)
    def _(g):
        plsc.addupdate_scatter(deg_v, [dst_v[pl.ds(g, L)]], ew_v[pl.ds(g, L)])

    pltpu.sync_copy(deg_v, out_hbm.at[_wid()])


def _make_sc_agg(F, G):
    n_pass = F // G

    @functools.partial(
        pl.kernel,
        out_type=jax.ShapeDtypeStruct((NW, F, N), jnp.float32),
        mesh=_mesh,
        compiler_params=_cp,
        name=f"sc_agg{F}",
        scratch_types=[
            pltpu.VMEM((N,), jnp.float32),
            pltpu.VMEM((EPW,), jnp.float32),
            pltpu.VMEM((EPW,), jnp.int32),
            pltpu.VMEM((EPW,), jnp.int32),
        ] + [pltpu.VMEM((N,), jnp.float32)] * (2 * G),
    )
    def _sc_agg(src_hbm, dst_hbm, ew_hbm, hT_hbm, dis_hbm, out_hbm,
                dis_v, nrm_v, src_v, dst_v, *rows):
        h_fs = rows[:G]
        acc_fs = rows[G:]
        base = _wid() * EPW
        pltpu.sync_copy(src_hbm.at[pl.ds(base, EPW)], src_v)
        pltpu.sync_copy(dst_hbm.at[pl.ds(base, EPW)], dst_v)
        pltpu.sync_copy(ew_hbm.at[pl.ds(base, EPW)], nrm_v)
        pltpu.sync_copy(dis_hbm, dis_v)

        @plsc.parallel_loop(0, EPW, step=L, unroll=4)
        def _(g):
            s16 = src_v[pl.ds(g, L)]
            d16 = dst_v[pl.ds(g, L)]
            dsv = plsc.load_gather(dis_v, [s16])
            ddv = plsc.load_gather(dis_v, [d16])
            nrm_v[pl.ds(g, L)] = dsv * nrm_v[pl.ds(g, L)] * ddv

        z16 = jnp.zeros((L,), jnp.float32)
        for p in range(n_pass):
            for f in range(G):
                pltpu.sync_copy(hT_hbm.at[p * G + f], h_fs[f])

            @plsc.parallel_loop(0, N, step=L, unroll=8)
            def _(i):
                for f in range(G):
                    acc_fs[f][pl.ds(i, L)] = z16

            @plsc.parallel_loop(0, EPW, step=L, unroll=4)
            def _(g):
                s16 = src_v[pl.ds(g, L)]
                d16 = dst_v[pl.ds(g, L)]
                n16 = nrm_v[pl.ds(g, L)]
                for f in range(G):
                    hf = plsc.load_gather(h_fs[f], [s16])
                    plsc.addupdate_scatter(acc_fs[f], [d16], hf * n16)

            for f in range(G):
                pltpu.sync_copy(acc_fs[f], out_hbm.at[_wid(), p * G + f])

    return _sc_agg


_sc_agg16 = _make_sc_agg(16, 4)
_sc_agg2 = _make_sc_agg(2, 2)


def _tc_h1T(W1, x):
    def body(w_ref, x_ref, o_ref):
        o_ref[...] = lax.dot_general(
            w_ref[...], x_ref[...], (((0,), (1,)), ((), ())),
            preferred_element_type=jnp.float32)

    return pl.pallas_call(
        body, out_shape=jax.ShapeDtypeStruct((16, N), jnp.float32),
    )(W1, x)


def _tc_dis(deg_part):
    def body(dp_ref, o_ref):
        deg = jnp.sum(dp_ref[...], axis=0, keepdims=True) + 1.0
        o_ref[...] = lax.rsqrt(deg)

    return pl.pallas_call(
        body, out_shape=jax.ShapeDtypeStruct((1, N), jnp.float32),
    )(deg_part)


def _tc_layer2_prep(agg1p, h1T, dis2d, b1c, W2):
    def body(a_ref, h_ref, d_ref, b_ref, w_ref, o_ref):
        aggsum = jnp.sum(a_ref[...], axis=0)
        dis2 = d_ref[...] * d_ref[...]
        out1 = aggsum + h_ref[...] * dis2 + b_ref[...]
        r = jnp.maximum(out1, 0.0)
        o_ref[...] = lax.dot_general(
            w_ref[...], r, (((0,), (0,)), ((), ())),
            preferred_element_type=jnp.float32)

    return pl.pallas_call(
        body, out_shape=jax.ShapeDtypeStruct((2, N), jnp.float32),
    )(agg1p, h1T, dis2d, b1c, W2)


def _tc_final(agg2p, h2T, dis2d, b2c):
    def body(a_ref, h_ref, d_ref, b_ref, o_ref):
        aggsum = jnp.sum(a_ref[...], axis=0)
        dis2 = d_ref[...] * d_ref[...]
        o2 = aggsum + h_ref[...] * dis2 + b_ref[...]
        m = jnp.max(o2, axis=0, keepdims=True)
        lse = m + jnp.log(jnp.sum(jnp.exp(o2 - m), axis=0, keepdims=True))
        o_ref[...] = o2 - lse

    return pl.pallas_call(
        body, out_shape=jax.ShapeDtypeStruct((2, N), jnp.float32),
    )(agg2p, h2T, dis2d, b2c)


@jax.jit
def kernel(x, edge_index, edge_weight, W1, b1, W2, b2):
    ei = edge_index.astype(jnp.int32)
    src = ei[0]
    dst = ei[1]

    deg_part = _sc_deg(dst, edge_weight)
    h1T = _tc_h1T(W1, x)
    dis2d = _tc_dis(deg_part)
    dis1d = dis2d.reshape(N)

    agg1p = _sc_agg16(src, dst, edge_weight, h1T, dis1d)
    h2T = _tc_layer2_prep(agg1p, h1T, dis2d, b1.reshape(16, 1), W2)
    agg2p = _sc_agg2(src, dst, edge_weight, h2T, dis1d)
    lsmT = _tc_final(agg2p, h2T, dis2d, b2.reshape(2, 1))
    return lsmT.T
